# baseline (speedup 1.0000x reference)
import functools

import jax
import jax.numpy as jnp
from jax.experimental import pallas as pl
from jax.experimental.pallas import tpu as pltpu
from jax.experimental.pallas import tpu_sc as plsc

_NC = 2
_NS = 16
_L = 16


def _router_body(logits_hbm, gs_hbm, row_v, gs_v):
    wid = jax.lax.axis_index("s") * _NC + jax.lax.axis_index("c")
    tpw = 2
    base = wid * tpw
    pltpu.sync_copy(logits_hbm.at[pl.ds(base, tpw)], row_v)
    nch = 64 // _L
    for t in range(tpw):
        chunks = [row_v[t, pl.ds(j * _L, _L)] for j in range(nch)]
        m = chunks[0]
        for c in chunks[1:]:
            m = jnp.maximum(m, c)
        m = jnp.max(m)
        ps = [jnp.exp(c - m) for c in chunks]
        s = ps[0]
        for p in ps[1:]:
            s = s + p
        s = jnp.sum(s)
        inv = 1.0 / jnp.broadcast_to(s, (_L,))
        for j in range(nch):
            gs_v[t, pl.ds(j * _L, _L)] = ps[j] * inv
    pltpu.sync_copy(gs_v, gs_hbm.at[pl.ds(base, tpw)])


def _experts_body(x_ref, gw_ref,
                  w1_ref, b1_ref, w2_ref, b2_ref, w3_ref, b3_ref,
                  out_ref, aux_ref, logits_ref,
                  i1_ref, i2_ref, tw_ref):
    e = pl.program_id(0)
    n_e = pl.num_programs(0)

    @pl.when(e == 0)
    def _gate():
        logits = jnp.dot(x_ref[...], gw_ref[...],
                         preferred_element_type=jnp.float32)
        logits_ref[...] = logits
        m = jnp.max(logits, axis=-1, keepdims=True)
        p = jnp.exp(logits - m)
        s = jnp.sum(p, axis=-1, keepdims=True)
        gs = p / s
        ids = jax.lax.broadcasted_iota(jnp.int32, gs.shape, 1)
        v1 = jnp.max(gs, axis=-1, keepdims=True)
        i1 = jnp.min(jnp.where(gs == v1, ids, n_e), axis=-1, keepdims=True)
        gs_m = jnp.where(ids == i1, -jnp.inf, gs)
        v2 = jnp.max(gs_m, axis=-1, keepdims=True)
        i2 = jnp.min(jnp.where(gs_m == v2, ids, n_e), axis=-1, keepdims=True)
        i1_ref[...] = i1
        i2_ref[...] = i2
        tw_ref[...] = v1 + v2
        usage = jnp.mean(gs, axis=0)
        lbl = -jnp.sum(usage * jnp.log(usage + 1e-9))
        lse = m[:, 0] + jnp.log(s[:, 0])
        z = jnp.mean(lse * lse) * 0.001
        aux_ref[...] = (lbl + z).reshape(1, 1)
        out_ref[...] = jnp.zeros_like(out_ref)

    xx = x_ref[...].astype(jnp.bfloat16)
    h1 = jnp.dot(xx, w1_ref[0].astype(jnp.bfloat16),
                 preferred_element_type=jnp.float32) + b1_ref[0]
    h2 = jnp.dot(xx, w2_ref[0].astype(jnp.bfloat16),
                 preferred_element_type=jnp.float32) + b2_ref[0]
    h = (h1 * jax.nn.sigmoid(h1)) * h2
    eo = jnp.dot(h.astype(jnp.bfloat16), w3_ref[0].astype(jnp.bfloat16),
                 preferred_element_type=jnp.float32) + b3_ref[0]
    w = jnp.where((i1_ref[...] == e) | (i2_ref[...] == e), tw_ref[...], 0.0)
    out_ref[...] += eo * w


def kernel(x, gate_w, W1, b1, W2, b2, W3, b3):
    B, S, D = x.shape
    E = gate_w.shape[1]
    H = W1.shape[2]
    T = B * S
    x2 = x.reshape(T, D)
    b1r = b1.reshape(E, 1, H)
    b2r = b2.reshape(E, 1, H)
    b3r = b3.reshape(E, 1, D)

    router = functools.partial(
        pl.kernel,
        mesh=plsc.VectorSubcoreMesh(core_axis_name="c", subcore_axis_name="s"),
        out_type=jax.ShapeDtypeStruct((T, E), jnp.float32),
        scratch_types=[pltpu.VMEM((2, E), jnp.float32),
                       pltpu.VMEM((2, E), jnp.float32)],
        compiler_params=pltpu.CompilerParams(needs_layout_passes=False),
    )(_router_body)

    out, aux, logits = pl.pallas_call(
        _experts_body,
        grid=(E,),
        in_specs=[
            pl.BlockSpec((T, D), lambda e: (0, 0)),
            pl.BlockSpec((D, E), lambda e: (0, 0)),
            pl.BlockSpec((1, D, H), lambda e: (e, 0, 0)),
            pl.BlockSpec((1, 1, H), lambda e: (e, 0, 0)),
            pl.BlockSpec((1, D, H), lambda e: (e, 0, 0)),
            pl.BlockSpec((1, 1, H), lambda e: (e, 0, 0)),
            pl.BlockSpec((1, H, D), lambda e: (e, 0, 0)),
            pl.BlockSpec((1, 1, D), lambda e: (e, 0, 0)),
        ],
        out_specs=[
            pl.BlockSpec((T, D), lambda e: (0, 0)),
            pl.BlockSpec((1, 1), lambda e: (0, 0)),
            pl.BlockSpec((T, E), lambda e: (0, 0)),
        ],
        out_shape=[
            jax.ShapeDtypeStruct((T, D), jnp.float32),
            jax.ShapeDtypeStruct((1, 1), jnp.float32),
            jax.ShapeDtypeStruct((T, E), jnp.float32),
        ],
        scratch_shapes=[
            pltpu.VMEM((T, 1), jnp.int32),
            pltpu.VMEM((T, 1), jnp.int32),
            pltpu.VMEM((T, 1), jnp.float32),
        ],
        compiler_params=pltpu.CompilerParams(
            dimension_semantics=("arbitrary",),
        ),
    )(x2, gate_w, W1, b1r, W2, b2r, W3, b3r)
    gs = router(logits)
    return out.reshape(B, S, D), aux[0, 0], gs.reshape(B, S, E)


# speedup vs baseline: 1.0231x; 1.0231x over previous
import functools

import jax
import jax.numpy as jnp
from jax.experimental import pallas as pl
from jax.experimental.pallas import tpu as pltpu
from jax.experimental.pallas import tpu_sc as plsc

_NC = 2
_NS = 16
_L = 16


def _router_body(logits_hbm, gs_hbm, row_v, gs_v):
    wid = jax.lax.axis_index("s") * _NC + jax.lax.axis_index("c")
    tpw = 2
    base = wid * tpw
    pltpu.sync_copy(logits_hbm.at[pl.ds(base, tpw)], row_v)
    nch = 64 // _L
    for t in range(tpw):
        chunks = [row_v[t, pl.ds(j * _L, _L)] for j in range(nch)]
        m = chunks[0]
        for c in chunks[1:]:
            m = jnp.maximum(m, c)
        m = jnp.max(m)
        ps = [jnp.exp(c - m) for c in chunks]
        s = ps[0]
        for p in ps[1:]:
            s = s + p
        s = jnp.sum(s)
        inv = 1.0 / jnp.broadcast_to(s, (_L,))
        for j in range(nch):
            gs_v[t, pl.ds(j * _L, _L)] = ps[j] * inv
    pltpu.sync_copy(gs_v, gs_hbm.at[pl.ds(base, tpw)])


def _experts_body(x_ref, gw_ref,
                  w1_ref, b1_ref, w2_ref, b2_ref, w3_ref, b3_ref,
                  out_ref, aux_ref, logits_ref,
                  i1_ref, i2_ref, tw_ref):
    e = pl.program_id(0)
    n_e = pl.num_programs(0)

    @pl.when(e == 0)
    def _gate():
        logits = jnp.dot(x_ref[...], gw_ref[...],
                         preferred_element_type=jnp.float32)
        logits_ref[...] = logits
        m = jnp.max(logits, axis=-1, keepdims=True)
        p = jnp.exp(logits - m)
        s = jnp.sum(p, axis=-1, keepdims=True)
        gs = p / s
        ids = jax.lax.broadcasted_iota(jnp.int32, gs.shape, 1)
        v1 = jnp.max(gs, axis=-1, keepdims=True)
        i1 = jnp.min(jnp.where(gs == v1, ids, n_e), axis=-1, keepdims=True)
        gs_m = jnp.where(ids == i1, -jnp.inf, gs)
        v2 = jnp.max(gs_m, axis=-1, keepdims=True)
        i2 = jnp.min(jnp.where(gs_m == v2, ids, n_e), axis=-1, keepdims=True)
        i1_ref[...] = i1
        i2_ref[...] = i2
        tw_ref[...] = v1 + v2
        usage = jnp.mean(gs, axis=0)
        lbl = -jnp.sum(usage * jnp.log(usage + 1e-9))
        lse = m[:, 0] + jnp.log(s[:, 0])
        z = jnp.mean(lse * lse) * 0.001
        aux_ref[...] = (lbl + z).reshape(1, 1)
        out_ref[...] = jnp.zeros_like(out_ref)

    xx = x_ref[...]
    h1 = (jnp.dot(xx, w1_ref[0], preferred_element_type=jnp.float32)
          + b1_ref[pl.ds(e, 1), 0])
    h2 = (jnp.dot(xx, w2_ref[0], preferred_element_type=jnp.float32)
          + b2_ref[pl.ds(e, 1), 0])
    h = (h1 * jax.nn.sigmoid(h1)) * h2
    eo = (jnp.dot(h, w3_ref[0], preferred_element_type=jnp.float32)
          + b3_ref[pl.ds(e, 1), 0])
    w = jnp.where((i1_ref[...] == e) | (i2_ref[...] == e), tw_ref[...], 0.0)
    out_ref[...] += eo * w


def kernel(x, gate_w, W1, b1, W2, b2, W3, b3):
    B, S, D = x.shape
    E = gate_w.shape[1]
    H = W1.shape[2]
    T = B * S
    x2 = x.reshape(T, D)
    b1r = b1.reshape(E, 1, H)
    b2r = b2.reshape(E, 1, H)
    b3r = b3.reshape(E, 1, D)

    router = functools.partial(
        pl.kernel,
        mesh=plsc.VectorSubcoreMesh(core_axis_name="c", subcore_axis_name="s"),
        out_type=jax.ShapeDtypeStruct((T, E), jnp.float32),
        scratch_types=[pltpu.VMEM((2, E), jnp.float32),
                       pltpu.VMEM((2, E), jnp.float32)],
        compiler_params=pltpu.CompilerParams(needs_layout_passes=False),
    )(_router_body)

    out, aux, logits = pl.pallas_call(
        _experts_body,
        grid=(E,),
        in_specs=[
            pl.BlockSpec((T, D), lambda e: (0, 0)),
            pl.BlockSpec((D, E), lambda e: (0, 0)),
            pl.BlockSpec((1, D, H), lambda e: (e, 0, 0)),
            pl.BlockSpec((E, 1, H), lambda e: (0, 0, 0)),
            pl.BlockSpec((1, D, H), lambda e: (e, 0, 0)),
            pl.BlockSpec((E, 1, H), lambda e: (0, 0, 0)),
            pl.BlockSpec((1, H, D), lambda e: (e, 0, 0)),
            pl.BlockSpec((E, 1, D), lambda e: (0, 0, 0)),
        ],
        out_specs=[
            pl.BlockSpec((T, D), lambda e: (0, 0)),
            pl.BlockSpec((1, 1), lambda e: (0, 0)),
            pl.BlockSpec((T, E), lambda e: (0, 0)),
        ],
        out_shape=[
            jax.ShapeDtypeStruct((T, D), jnp.float32),
            jax.ShapeDtypeStruct((1, 1), jnp.float32),
            jax.ShapeDtypeStruct((T, E), jnp.float32),
        ],
        scratch_shapes=[
            pltpu.VMEM((T, 1), jnp.int32),
            pltpu.VMEM((T, 1), jnp.int32),
            pltpu.VMEM((T, 1), jnp.float32),
        ],
        compiler_params=pltpu.CompilerParams(
            dimension_semantics=("arbitrary",),
        ),
    )(x2, gate_w, W1, b1r, W2, b2r, W3, b3r)
    gs = router(logits)
    return out.reshape(B, S, D), aux[0, 0], gs.reshape(B, S, E)


# speedup vs baseline: 1.0382x; 1.0148x over previous
import functools

import jax
import jax.numpy as jnp
from jax.experimental import pallas as pl
from jax.experimental.pallas import tpu as pltpu
from jax.experimental.pallas import tpu_sc as plsc

_NC = 2
_NS = 16
_L = 16


def _router_body(logits_hbm, gs_hbm, row_v, gs_v):
    wid = jax.lax.axis_index("s")
    tpw = 4
    base = wid * tpw
    pltpu.sync_copy(logits_hbm.at[pl.ds(base, tpw)], row_v)
    nch = 64 // _L
    for t in range(tpw):
        chunks = [row_v[t, pl.ds(j * _L, _L)] for j in range(nch)]
        m = chunks[0]
        for c in chunks[1:]:
            m = jnp.maximum(m, c)
        m = jnp.max(m)
        ps = [jnp.exp(c - m) for c in chunks]
        s = ps[0]
        for p in ps[1:]:
            s = s + p
        s = jnp.sum(s)
        inv = 1.0 / jnp.broadcast_to(s, (_L,))
        for j in range(nch):
            gs_v[t, pl.ds(j * _L, _L)] = ps[j] * inv
    pltpu.sync_copy(gs_v, gs_hbm.at[pl.ds(base, tpw)])


def _experts_body(x_ref, gw_ref,
                  w1_ref, b1_ref, w2_ref, b2_ref, w3_ref, b3_ref,
                  out_ref, aux_ref, logits_ref,
                  i1_ref, i2_ref, tw_ref):
    e = pl.program_id(0)
    n_e = pl.num_programs(0)

    @pl.when(e == 0)
    def _gate():
        logits = jnp.dot(x_ref[...], gw_ref[...],
                         preferred_element_type=jnp.float32)
        logits_ref[...] = logits
        m = jnp.max(logits, axis=-1, keepdims=True)
        p = jnp.exp(logits - m)
        s = jnp.sum(p, axis=-1, keepdims=True)
        gs = p / s
        ids = jax.lax.broadcasted_iota(jnp.int32, gs.shape, 1)
        v1 = jnp.max(gs, axis=-1, keepdims=True)
        i1 = jnp.min(jnp.where(gs == v1, ids, n_e), axis=-1, keepdims=True)
        gs_m = jnp.where(ids == i1, -jnp.inf, gs)
        v2 = jnp.max(gs_m, axis=-1, keepdims=True)
        i2 = jnp.min(jnp.where(gs_m == v2, ids, n_e), axis=-1, keepdims=True)
        i1_ref[...] = i1
        i2_ref[...] = i2
        tw_ref[...] = v1 + v2
        usage = jnp.mean(gs, axis=0)
        lbl = -jnp.sum(usage * jnp.log(usage + 1e-9))
        lse = m[:, 0] + jnp.log(s[:, 0])
        z = jnp.mean(lse * lse) * 0.001
        aux_ref[...] = (lbl + z).reshape(1, 1)
        out_ref[...] = jnp.zeros_like(out_ref)

    xx = x_ref[...]
    h1 = (jnp.dot(xx, w1_ref[0], preferred_element_type=jnp.float32)
          + b1_ref[pl.ds(e, 1), 0])
    h2 = (jnp.dot(xx, w2_ref[0], preferred_element_type=jnp.float32)
          + b2_ref[pl.ds(e, 1), 0])
    h = (h1 * jax.nn.sigmoid(h1)) * h2
    eo = (jnp.dot(h, w3_ref[0], preferred_element_type=jnp.float32)
          + b3_ref[pl.ds(e, 1), 0])
    w = jnp.where((i1_ref[...] == e) | (i2_ref[...] == e), tw_ref[...], 0.0)
    out_ref[...] += eo * w


def kernel(x, gate_w, W1, b1, W2, b2, W3, b3):
    B, S, D = x.shape
    E = gate_w.shape[1]
    H = W1.shape[2]
    T = B * S
    x2 = x.reshape(T, D)
    b1r = b1.reshape(E, 1, H)
    b2r = b2.reshape(E, 1, H)
    b3r = b3.reshape(E, 1, D)

    router = functools.partial(
        pl.kernel,
        mesh=plsc.VectorSubcoreMesh(core_axis_name="c", subcore_axis_name="s",
                                    num_cores=1),
        out_type=jax.ShapeDtypeStruct((T, E), jnp.float32),
        scratch_types=[pltpu.VMEM((4, E), jnp.float32),
                       pltpu.VMEM((4, E), jnp.float32)],
        compiler_params=pltpu.CompilerParams(needs_layout_passes=False),
    )(_router_body)

    out, aux, logits = pl.pallas_call(
        _experts_body,
        grid=(E,),
        in_specs=[
            pl.BlockSpec((T, D), lambda e: (0, 0)),
            pl.BlockSpec((D, E), lambda e: (0, 0)),
            pl.BlockSpec((1, D, H), lambda e: (e, 0, 0)),
            pl.BlockSpec((E, 1, H), lambda e: (0, 0, 0)),
            pl.BlockSpec((1, D, H), lambda e: (e, 0, 0)),
            pl.BlockSpec((E, 1, H), lambda e: (0, 0, 0)),
            pl.BlockSpec((1, H, D), lambda e: (e, 0, 0)),
            pl.BlockSpec((E, 1, D), lambda e: (0, 0, 0)),
        ],
        out_specs=[
            pl.BlockSpec((T, D), lambda e: (0, 0)),
            pl.BlockSpec((1, 1), lambda e: (0, 0)),
            pl.BlockSpec((T, E), lambda e: (0, 0)),
        ],
        out_shape=[
            jax.ShapeDtypeStruct((T, D), jnp.float32),
            jax.ShapeDtypeStruct((1, 1), jnp.float32),
            jax.ShapeDtypeStruct((T, E), jnp.float32),
        ],
        scratch_shapes=[
            pltpu.VMEM((T, 1), jnp.int32),
            pltpu.VMEM((T, 1), jnp.int32),
            pltpu.VMEM((T, 1), jnp.float32),
        ],
        compiler_params=pltpu.CompilerParams(
            dimension_semantics=("arbitrary",),
        ),
    )(x2, gate_w, W1, b1r, W2, b2r, W3, b3r)
    gs = router(logits)
    return out.reshape(B, S, D), aux[0, 0], gs.reshape(B, S, E)
